# baseline (speedup 1.0000x reference)
import functools

import jax
import jax.numpy as jnp
from jax.experimental import pallas as pl
from jax.experimental.pallas import tpu as pltpu

H = 4
DK = 32
HID = H * DK
B = 64
R = 512


def _fused_kernel(m, mpad,
                  x_ref, bt_ref, w_ref,
                  base_ref, trip_ref,
                  out_ref, xp_ref, xt_ref, yt_ref, et_ref, bti_ref, wb_ref):
    xp_ref[0:m, :] = x_ref[...]
    xp_ref[m:mpad, :] = jnp.zeros((mpad - m, HID), jnp.float32)
    xt_ref[:, 0:m] = x_ref[...].T
    xt_ref[:, m:mpad] = jnp.zeros((HID, mpad - m), jnp.float32)
    bti_ref[:, 0:m] = bt_ref[...]
    bti_ref[:, m:mpad] = jnp.full((1, mpad - m), B, jnp.int32)

    wb_ref[...] = jnp.zeros((HID, HID), jnp.float32)
    for h in range(H):
        wh = jnp.concatenate(
            [w_ref[h:h + 1, i * DK:(i + 1) * DK] for i in range(DK)],
            axis=0)
        wb_ref[h * DK:(h + 1) * DK, h * DK:(h + 1) * DK] = wh.T

    xt = xt_ref[...]
    tt = jnp.dot(wb_ref[...], xt, preferred_element_type=jnp.float32)
    pt = tt * xt
    s4 = jnp.concatenate(
        [jnp.sum(pt[h * DK:(h + 1) * DK, :], axis=0, keepdims=True)
         for h in range(H)], axis=0)
    gm = jnp.max(s4, axis=1, keepdims=True)
    et = jnp.exp(s4 - gm)
    et_ref[...] = et
    e128 = jnp.concatenate(
        [jnp.broadcast_to(et[h:h + 1, :], (DK, mpad)) for h in range(H)],
        axis=0)
    yt_ref[...] = e128 * xt

    for b in range(B):
        base = base_ref[b]
        trip = trip_ref[b]

        def chunk_body(j, c, b=b):
            gram, dsv = c
            k = pl.multiple_of((base + j) * R, 128)
            msk = (bti_ref[:, pl.ds(k, R)] == b).astype(jnp.float32)
            ym = yt_ref[:, pl.ds(k, R)] * msk
            gram = gram + jnp.dot(ym, xp_ref[pl.ds(k, R), :],
                                  preferred_element_type=jnp.float32)
            dsv = dsv + et_ref[:, pl.ds(k, R)] * msk
            return gram, dsv

        gram, dsv = jax.lax.fori_loop(
            0, trip, chunk_body,
            (jnp.zeros((HID, HID), jnp.float32), jnp.zeros((H, R), jnp.float32)))
        dsum = jnp.sum(dsv, axis=1, keepdims=True)
        rec = 1.0 / (dsum + 1e-16)
        for h in range(H):
            blk = gram[h * DK:(h + 1) * DK, h * DK:(h + 1) * DK]
            out_ref[h, b] = blk * rec[h:h + 1, 0:1]


def kernel(x, batch, edge, attn_W, attn_b):
    del edge, attn_b
    m = x.shape[0]
    mpad = ((m + R - 1) // R) * R

    xf = x.astype(jnp.float32)
    bi = batch.astype(jnp.int32)
    bt = bi[None, :]
    wf = attn_W.astype(jnp.float32)

    segs = jnp.arange(B, dtype=jnp.int32)
    starts = jnp.searchsorted(bi, segs, side="left")
    ends = jnp.concatenate([starts[1:], jnp.array([m], starts.dtype)])
    base = (starts // R).astype(jnp.int32)
    trip = jnp.where(ends > starts,
                     (ends - 1) // R - starts // R + 1, 0).astype(jnp.int32)

    out4 = pl.pallas_call(
        functools.partial(_fused_kernel, m, mpad),
        in_specs=[
            pl.BlockSpec(memory_space=pltpu.VMEM),
            pl.BlockSpec(memory_space=pltpu.VMEM),
            pl.BlockSpec(memory_space=pltpu.VMEM),
            pl.BlockSpec(memory_space=pltpu.SMEM),
            pl.BlockSpec(memory_space=pltpu.SMEM),
        ],
        out_shape=jax.ShapeDtypeStruct((H, B, DK, DK), jnp.float32),
        scratch_shapes=[pltpu.VMEM((mpad, HID), jnp.float32),
                        pltpu.VMEM((HID, mpad), jnp.float32),
                        pltpu.VMEM((HID, mpad), jnp.float32),
                        pltpu.VMEM((H, mpad), jnp.float32),
                        pltpu.VMEM((1, mpad), jnp.int32),
                        pltpu.VMEM((HID, HID), jnp.float32)],
    )(xf, bt, wf, base, trip)

    return out4.reshape(B, H * DK * DK)


# speedup vs baseline: 1.4186x; 1.4186x over previous
import functools

import jax
import jax.numpy as jnp
from jax.experimental import pallas as pl
from jax.experimental.pallas import tpu as pltpu

H = 4
DK = 32
HID = H * DK
B = 64
R = 512


def _fused_kernel(m, mpad,
                  x_ref, bt_ref, w_ref,
                  base_ref, trip_ref,
                  out_ref, xp_ref, xt_ref, yt_ref, et_ref, bti_ref, wb_ref):
    xp_ref[0:m, :] = x_ref[...]
    xp_ref[m:mpad, :] = jnp.zeros((mpad - m, HID), jnp.float32)
    xt_ref[:, 0:m] = x_ref[...].T
    xt_ref[:, m:mpad] = jnp.zeros((HID, mpad - m), jnp.float32)
    bti_ref[:, 0:m] = bt_ref[...]
    bti_ref[:, m:mpad] = jnp.full((1, mpad - m), B, jnp.int32)

    wb_ref[...] = jnp.zeros((HID, HID), jnp.float32)
    for h in range(H):
        wh = jnp.concatenate(
            [w_ref[h:h + 1, i * DK:(i + 1) * DK] for i in range(DK)],
            axis=0)
        wb_ref[h * DK:(h + 1) * DK, h * DK:(h + 1) * DK] = wh.T

    xt = xt_ref[...]
    tt = jnp.dot(wb_ref[...], xt, preferred_element_type=jnp.float32)
    pt = tt * xt
    s4 = jnp.concatenate(
        [jnp.sum(pt[h * DK:(h + 1) * DK, :], axis=0, keepdims=True)
         for h in range(H)], axis=0)
    gm = jnp.max(s4, axis=1, keepdims=True)
    et = jnp.exp(s4 - gm)
    et_ref[...] = et
    e128 = jnp.concatenate(
        [jnp.broadcast_to(et[h:h + 1, :], (DK, mpad)) for h in range(H)],
        axis=0)
    yt_ref[...] = e128 * xt

    for b in range(B):
        base = base_ref[b]
        trip = trip_ref[b]

        def chunk_body(j, c, b=b):
            gram, dsv = c
            k = pl.multiple_of((base + j) * R, 128)
            msk = (bti_ref[:, pl.ds(k, R)] == b).astype(jnp.float32)
            ym = yt_ref[:, pl.ds(k, R)] * msk
            gram = gram + jnp.dot(ym, xp_ref[pl.ds(k, R), :],
                                  preferred_element_type=jnp.float32)
            dsv = dsv + et_ref[:, pl.ds(k, R)] * msk
            return gram, dsv

        gram, dsv = jax.lax.fori_loop(
            0, trip, chunk_body,
            (jnp.zeros((HID, HID), jnp.float32), jnp.zeros((H, R), jnp.float32)))
        dsum = jnp.sum(dsv, axis=1, keepdims=True)
        rec = 1.0 / (dsum + 1e-16)
        for h in range(H):
            blk = gram[h * DK:(h + 1) * DK, h * DK:(h + 1) * DK]
            out_ref[h, b] = blk * rec[h:h + 1, 0:1]


def kernel(x, batch, edge, attn_W, attn_b):
    del edge, attn_b
    m = x.shape[0]
    mpad = ((m + R - 1) // R) * R

    xf = x.astype(jnp.float32)
    bi = batch.astype(jnp.int32)
    bt = bi[None, :]
    wf = attn_W.astype(jnp.float32)

    segs = jnp.arange(B + 1, dtype=jnp.int32)
    bounds = jnp.sum(bi[None, :] < segs[:, None], axis=1)
    starts = bounds[:B]
    ends = bounds[1:]
    base = (starts // R).astype(jnp.int32)
    trip = jnp.where(ends > starts,
                     (ends - 1) // R - starts // R + 1, 0).astype(jnp.int32)

    out4 = pl.pallas_call(
        functools.partial(_fused_kernel, m, mpad),
        in_specs=[
            pl.BlockSpec(memory_space=pltpu.VMEM),
            pl.BlockSpec(memory_space=pltpu.VMEM),
            pl.BlockSpec(memory_space=pltpu.VMEM),
            pl.BlockSpec(memory_space=pltpu.SMEM),
            pl.BlockSpec(memory_space=pltpu.SMEM),
        ],
        out_shape=jax.ShapeDtypeStruct((H, B, DK, DK), jnp.float32),
        scratch_shapes=[pltpu.VMEM((mpad, HID), jnp.float32),
                        pltpu.VMEM((HID, mpad), jnp.float32),
                        pltpu.VMEM((HID, mpad), jnp.float32),
                        pltpu.VMEM((H, mpad), jnp.float32),
                        pltpu.VMEM((1, mpad), jnp.int32),
                        pltpu.VMEM((HID, HID), jnp.float32)],
    )(xf, bt, wf, base, trip)

    return out4.reshape(B, H * DK * DK)


# speedup vs baseline: 1.9037x; 1.3419x over previous
import functools

import jax
import jax.numpy as jnp
from jax.experimental import pallas as pl
from jax.experimental.pallas import tpu as pltpu

H = 4
DK = 32
HID = H * DK
B = 64
R = 512


def _fused_kernel(m, mpad,
                  x_ref, bt_ref, w_ref,
                  base_ref, trip_ref,
                  out_ref, xp_ref, xt_ref, yt_ref, et_ref, bti_ref, wb_ref):
    xp_ref[0:m, :] = x_ref[...]
    xp_ref[m:mpad, :] = jnp.zeros((mpad - m, HID), jnp.float32)
    xt_ref[:, 0:m] = x_ref[...].T
    xt_ref[:, m:mpad] = jnp.zeros((HID, mpad - m), jnp.float32)
    bti_ref[:, 0:m] = bt_ref[...]
    bti_ref[:, m:mpad] = jnp.full((1, mpad - m), B, jnp.int32)

    wb_ref[...] = jnp.zeros((HID, HID), jnp.float32)
    for h in range(H):
        wh = jnp.concatenate(
            [w_ref[h:h + 1, i * DK:(i + 1) * DK] for i in range(DK)],
            axis=0)
        wb_ref[h * DK:(h + 1) * DK, h * DK:(h + 1) * DK] = wh.T

    xt = xt_ref[...]
    tt = jnp.dot(wb_ref[...], xt, preferred_element_type=jnp.float32)
    pt = tt * xt
    s4 = jnp.concatenate(
        [jnp.sum(pt[h * DK:(h + 1) * DK, :], axis=0, keepdims=True)
         for h in range(H)], axis=0)
    gm = jnp.max(s4, axis=1, keepdims=True)
    et = jnp.exp(s4 - gm)
    et_ref[...] = et
    e128 = jnp.concatenate(
        [jnp.broadcast_to(et[h:h + 1, :], (DK, mpad)) for h in range(H)],
        axis=0)
    yt_ref[...] = e128 * xt

    for b in range(B):
        base = base_ref[b]
        trip = trip_ref[b]

        def chunk_body(j, c, b=b):
            gram, dsv = c
            k = pl.multiple_of((base + j) * R, 128)
            msk = (bti_ref[:, pl.ds(k, R)] == b).astype(jnp.float32)
            ym = yt_ref[:, pl.ds(k, R)] * msk
            gram = gram + jnp.dot(ym, xp_ref[pl.ds(k, R), :],
                                  preferred_element_type=jnp.float32)
            dsv = dsv + et_ref[:, pl.ds(k, R)] * msk
            return gram, dsv

        c1 = chunk_body(0, (jnp.zeros((HID, HID), jnp.float32),
                            jnp.zeros((H, R), jnp.float32)))
        gram, dsv = jax.lax.fori_loop(1, trip, chunk_body, c1)
        dsum = jnp.sum(dsv, axis=1, keepdims=True)
        rec = 1.0 / (dsum + 1e-16)
        for h in range(H):
            blk = gram[h * DK:(h + 1) * DK, h * DK:(h + 1) * DK]
            out_ref[h, b] = blk * rec[h:h + 1, 0:1]


def kernel(x, batch, edge, attn_W, attn_b):
    del edge, attn_b
    m = x.shape[0]
    mpad = ((m + R - 1) // R) * R

    xf = x.astype(jnp.float32)
    bi = batch.astype(jnp.int32)
    bt = bi[None, :]
    wf = attn_W.astype(jnp.float32)

    segs = jnp.arange(B + 1, dtype=jnp.int32)
    bounds = jnp.sum(bi[None, :] < segs[:, None], axis=1)
    starts = bounds[:B]
    ends = bounds[1:]
    base = (starts // R).astype(jnp.int32)
    trip = jnp.where(ends > starts,
                     (ends - 1) // R - starts // R + 1, 0).astype(jnp.int32)

    out4 = pl.pallas_call(
        functools.partial(_fused_kernel, m, mpad),
        in_specs=[
            pl.BlockSpec(memory_space=pltpu.VMEM),
            pl.BlockSpec(memory_space=pltpu.VMEM),
            pl.BlockSpec(memory_space=pltpu.VMEM),
            pl.BlockSpec(memory_space=pltpu.SMEM),
            pl.BlockSpec(memory_space=pltpu.SMEM),
        ],
        out_shape=jax.ShapeDtypeStruct((H, B, DK, DK), jnp.float32),
        scratch_shapes=[pltpu.VMEM((mpad, HID), jnp.float32),
                        pltpu.VMEM((HID, mpad), jnp.float32),
                        pltpu.VMEM((HID, mpad), jnp.float32),
                        pltpu.VMEM((H, mpad), jnp.float32),
                        pltpu.VMEM((1, mpad), jnp.int32),
                        pltpu.VMEM((HID, HID), jnp.float32)],
    )(xf, bt, wf, base, trip)

    return out4.reshape(B, H * DK * DK)


# speedup vs baseline: 1.9389x; 1.0185x over previous
import functools

import jax
import jax.numpy as jnp
from jax.experimental import pallas as pl
from jax.experimental.pallas import tpu as pltpu

H = 4
DK = 32
HID = H * DK
B = 64
R = 512


def _fused_kernel(m, mpad,
                  x_ref, bt_ref, w_ref,
                  base_ref, trip_ref,
                  out_ref, xp_ref, xt_ref, yt_ref, et_ref, bti_ref, wb_ref):
    xp_ref[0:m, :] = x_ref[...]
    xp_ref[m:mpad, :] = jnp.zeros((mpad - m, HID), jnp.float32)
    xt_ref[:, 0:m] = x_ref[...].T
    xt_ref[:, m:mpad] = jnp.zeros((HID, mpad - m), jnp.float32)
    bti_ref[:, 0:m] = bt_ref[...]
    bti_ref[:, m:mpad] = jnp.full((1, mpad - m), B, jnp.int32)

    wb_ref[...] = jnp.zeros((HID, HID), jnp.float32)
    for h in range(H):
        wh = jnp.concatenate(
            [w_ref[h:h + 1, i * DK:(i + 1) * DK] for i in range(DK)],
            axis=0)
        wb_ref[h * DK:(h + 1) * DK, h * DK:(h + 1) * DK] = wh.T

    xt = xt_ref[...]
    tt = jnp.dot(wb_ref[...], xt, preferred_element_type=jnp.float32)
    pt = tt * xt
    s4 = jnp.concatenate(
        [jnp.sum(pt[h * DK:(h + 1) * DK, :], axis=0, keepdims=True)
         for h in range(H)], axis=0)
    gm = jnp.max(s4, axis=1, keepdims=True)
    et = jnp.exp(s4 - gm)
    et_ref[...] = et
    e128 = jnp.concatenate(
        [jnp.broadcast_to(et[h:h + 1, :], (DK, mpad)) for h in range(H)],
        axis=0)
    yt_ref[...] = e128 * xt

    for b in range(B):
        base = base_ref[b]
        trip = trip_ref[b]

        def chunk_body(j, c, b=b):
            gram, dsv = c
            k = pl.multiple_of((base + j) * R, 128)
            msk = (bti_ref[:, pl.ds(k, R)] == b).astype(jnp.float32)
            ym = yt_ref[:, pl.ds(k, R)] * msk
            gram = gram + jnp.dot(ym, xp_ref[pl.ds(k, R), :],
                                  preferred_element_type=jnp.float32)
            dsv = dsv + et_ref[:, pl.ds(k, R)] * msk
            return gram, dsv

        c1 = chunk_body(0, (jnp.zeros((HID, HID), jnp.float32),
                            jnp.zeros((H, R), jnp.float32)))
        k2 = pl.multiple_of(jnp.minimum(base + 1, mpad // R - 1) * R, 128)
        flag = (trip >= 2).astype(jnp.float32)
        mf2 = (bti_ref[:, pl.ds(k2, R)] == b).astype(jnp.float32) * flag
        ym2 = yt_ref[:, pl.ds(k2, R)] * mf2
        g2 = jnp.dot(ym2, xp_ref[pl.ds(k2, R), :],
                     preferred_element_type=jnp.float32)
        c2 = (c1[0] + g2, c1[1] + et_ref[:, pl.ds(k2, R)] * mf2)
        gram, dsv = jax.lax.fori_loop(2, trip, chunk_body, c2)
        dsum = jnp.sum(dsv, axis=1, keepdims=True)
        rec = 1.0 / (dsum + 1e-16)
        for h in range(H):
            blk = gram[h * DK:(h + 1) * DK, h * DK:(h + 1) * DK]
            out_ref[h, b] = blk * rec[h:h + 1, 0:1]


def kernel(x, batch, edge, attn_W, attn_b):
    del edge, attn_b
    m = x.shape[0]
    mpad = ((m + R - 1) // R) * R

    xf = x.astype(jnp.float32)
    bi = batch.astype(jnp.int32)
    bt = bi[None, :]
    wf = attn_W.astype(jnp.float32)

    segs = jnp.arange(B + 1, dtype=jnp.int32)
    bounds = jnp.sum(bi[None, :] < segs[:, None], axis=1)
    starts = bounds[:B]
    ends = bounds[1:]
    base = (starts // R).astype(jnp.int32)
    trip = jnp.where(ends > starts,
                     (ends - 1) // R - starts // R + 1, 0).astype(jnp.int32)

    out4 = pl.pallas_call(
        functools.partial(_fused_kernel, m, mpad),
        in_specs=[
            pl.BlockSpec(memory_space=pltpu.VMEM),
            pl.BlockSpec(memory_space=pltpu.VMEM),
            pl.BlockSpec(memory_space=pltpu.VMEM),
            pl.BlockSpec(memory_space=pltpu.SMEM),
            pl.BlockSpec(memory_space=pltpu.SMEM),
        ],
        out_shape=jax.ShapeDtypeStruct((H, B, DK, DK), jnp.float32),
        scratch_shapes=[pltpu.VMEM((mpad, HID), jnp.float32),
                        pltpu.VMEM((HID, mpad), jnp.float32),
                        pltpu.VMEM((HID, mpad), jnp.float32),
                        pltpu.VMEM((H, mpad), jnp.float32),
                        pltpu.VMEM((1, mpad), jnp.int32),
                        pltpu.VMEM((HID, HID), jnp.float32)],
    )(xf, bt, wf, base, trip)

    return out4.reshape(B, H * DK * DK)
